# baseline (speedup 1.0000x reference)
import functools

import jax
import jax.numpy as jnp
from jax.experimental import pallas as pl

_DROP_PROB = 0.0
_NUM_GRAPHS = 256
_ROWS = 1000


def _body(batch_ref, mask_ref, x_ref, o_ref):
    ids = batch_ref[0, 0, :]
    iota = jax.lax.broadcasted_iota(jnp.int32, (_ROWS, _NUM_GRAPHS), 1)
    onehot = (ids[:, None] == iota).astype(jnp.float32)
    scale = jnp.sum(onehot * mask_ref[0, :][None, :], axis=1, keepdims=True)
    o_ref[...] = x_ref[...] * scale


@functools.partial(jax.jit, static_argnames=())
def kernel(x, batch):
    n, d = x.shape
    num_blocks = n // _ROWS
    keep_prob = 1.0 - _DROP_PROB
    rnd = jax.random.uniform(jax.random.key(42), (_NUM_GRAPHS,), dtype=x.dtype)
    keep_mask = (jnp.floor(keep_prob + rnd) / keep_prob).reshape(1, _NUM_GRAPHS)

    batch3 = batch.reshape(num_blocks, 1, _ROWS)

    return pl.pallas_call(
        _body,
        grid=(num_blocks,),
        in_specs=[
            pl.BlockSpec((1, 1, _ROWS), lambda i: (i, 0, 0)),
            pl.BlockSpec((1, _NUM_GRAPHS), lambda i: (0, 0)),
            pl.BlockSpec((_ROWS, d), lambda i: (i, 0)),
        ],
        out_specs=pl.BlockSpec((_ROWS, d), lambda i: (i, 0)),
        out_shape=jax.ShapeDtypeStruct((n, d), x.dtype),
    )(batch3, keep_mask, x)
